# baseline (speedup 1.0000x reference)
import functools
import jax
import jax.numpy as jnp
from jax import lax
from jax.experimental import pallas as pl
from jax.experimental.pallas import tpu as pltpu
from jax.experimental.pallas import tpu_sc as plsc

N_NODES = 10000
D = 128
E = 320000
HID = 64

NC, NS = 2, 16
NW = NC * NS
BLK = 64
NB = E // BLK
NBW = NB // NW
NXTRA = NB - NBW * NW
UNROLL = 12
NSUP = NBW // UNROLL
CH = 40
NCHUNK = N_NODES // CH


def _emb_body(attr_ref, w_ref, b_ref, out_ref):
    out_ref[...] = (
        jnp.dot(attr_ref[...], w_ref[...], preferred_element_type=jnp.float32)
        + b_ref[...]
    )


def _edge_emb(attr128, w_big, b_big):
    rb = 800
    er = E // 8
    return pl.pallas_call(
        _emb_body,
        grid=(er // rb,),
        in_specs=[
            pl.BlockSpec((rb, D), lambda i: (i, 0)),
            pl.BlockSpec((D, 8 * D), lambda i: (0, 0)),
            pl.BlockSpec((1, 8 * D), lambda i: (0, 0)),
        ],
        out_specs=pl.BlockSpec((rb, 8 * D), lambda i: (i, 0)),
        out_shape=jax.ShapeDtypeStruct((er, 8 * D), jnp.float32),
    )(attr128, w_big, b_big)


def _sc_body(x_hbm, emb_hbm, src2_hbm, dst2_hbm, out_hbm,
             rows0, rows1, rows2, emb0, emb1,
             is0, is1, is2, id0, id1, id2, id3,
             accum,
             sg0, sg1, sg2, se0, se1, ss0, ss1, si0, si1, si2):
    cid = lax.axis_index("c")
    sid = lax.axis_index("s")
    wid = sid * NC + cid
    rows = (rows0, rows1, rows2)
    embv = (emb0, emb1)
    isrc = (is0, is1, is2)
    idst = (id0, id1, id2, id3)
    sg = (sg0, sg1, sg2)
    se = (se0, se1)
    ss = (ss0, ss1)
    si = (si0, si1, si2)

    zv = jnp.zeros((16,), jnp.float32)

    def zero_body(i, _):
        for j in range(D // 16):
            rows0[i, pl.ds(j * 16, 16)] = zv
        return 0

    lax.fori_loop(0, CH, zero_body, 0)
    nch = jnp.where(sid < NCHUNK - (NCHUNK // NS) * NS, NCHUNK // NS + 1,
                    NCHUNK // NS)

    def zchunk(k, _):
        r0 = pl.multiple_of((sid + k * NS) * CH, 8)
        pltpu.sync_copy(rows0.at[pl.ds(0, CH)], accum.at[pl.ds(r0, CH)])
        return 0

    lax.fori_loop(0, nch, zchunk, 0)
    plsc.subcore_barrier()


    def fire_idx(t, k3, k4):
        blk = t * NW + wid
        pltpu.async_copy(src2_hbm.at[blk], isrc[k3], si[k3])
        pltpu.async_copy(dst2_hbm.at[blk], idst[k4], si[k3])

    def drain_idx(k3):
        pltpu.make_async_copy(src2_hbm.at[0], isrc[k3], si[k3]).wait()
        pltpu.make_async_copy(dst2_hbm.at[0], idst[0], si[k3]).wait()

    def ebase(t):
        return pl.multiple_of((t * NW + wid) * (BLK // 8), 8)

    def fire_gather(t, k3, k2):
        pltpu.async_copy(x_hbm.at[isrc[k3]], rows[k3], sg[k3])
        pltpu.async_copy(emb_hbm.at[pl.ds(ebase(t), BLK // 8)], embv[k2], se[k2])

    def wait_gather(t, k3, k2):
        pltpu.make_async_copy(x_hbm.at[isrc[k3]], rows[k3], sg[k3]).wait()
        pltpu.make_async_copy(emb_hbm.at[pl.ds(ebase(t), BLK // 8)], embv[k2],
                              se[k2]).wait()

    def compute(k3, k2):
        def body(r, _):
            for c in range(8):
                for j in range(D // 16):
                    sl = pl.ds(j * 16, 16)
                    rows[k3][8 * r + c, sl] = jnp.maximum(
                        rows[k3][8 * r + c, sl]
                        + embv[k2][r, pl.ds(c * D + j * 16, 16)], 0.0)
            return 0

        lax.fori_loop(0, BLK // 8, body, 0)

    fire_idx(0, 0, 0)
    fire_idx(1, 1, 1)
    fire_idx(2, 2, 2)
    drain_idx(0)
    fire_gather(0, 0, 0)
    drain_idx(1)
    fire_gather(1, 1, 1)

    def super_body(s, _):
        for u in range(UNROLL):
            t = s * UNROLL + u
            k3, k2, k4 = u % 3, u % 2, u % 4
            wait_gather(t, k3, k2)
            compute(k3, k2)
            pltpu.async_copy(rows[k3], accum.at[idst[k4]], ss[u % 2], add=True)

            def drain_scat():
                pltpu.make_async_copy(rows[(u - 1) % 3],
                                      accum.at[idst[(u - 1) % 4]],
                                      ss[(u - 1) % 2]).wait()

            if u == 0:
                pl.when(s > 0)(drain_scat)
            else:
                drain_scat()

            @pl.when(t + 2 < NBW)
            def _():
                drain_idx((u + 2) % 3)
                fire_gather(t + 2, (u + 2) % 3, k2)

            @pl.when(t + 3 < NBW)
            def _():
                fire_idx(t + 3, u % 3, (u + 3) % 4)
        return 0

    lax.fori_loop(0, NSUP, super_body, 0)
    pltpu.make_async_copy(rows[(NBW - 1) % 3], accum.at[idst[(NBW - 1) % 4]],
                          ss[(NBW - 1) % 2]).wait()

    @pl.when(wid < NXTRA)
    def _():
        blk = NBW * NW + wid
        pltpu.sync_copy(src2_hbm.at[blk], isrc[0])
        pltpu.sync_copy(dst2_hbm.at[blk], idst[0])
        pltpu.async_copy(x_hbm.at[isrc[0]], rows[0], sg[0]).wait()
        pltpu.sync_copy(emb_hbm.at[pl.ds(ebase(NBW), BLK // 8)], embv[0])
        compute(0, 0)
        pltpu.sync_copy(rows[0], accum.at[idst[0]], add=True)

    plsc.subcore_barrier()

    def wchunk(k, _):
        r0 = pl.multiple_of((sid + k * NS) * CH, 8)
        pltpu.sync_copy(accum.at[pl.ds(r0, CH)], out_hbm.at[cid, pl.ds(r0, CH)])
        return 0

    lax.fori_loop(0, nch, wchunk, 0)


def _sc_aggregate(x, emb, src2, dst2):
    mesh = plsc.VectorSubcoreMesh(core_axis_name="c", subcore_axis_name="s")
    f = pl.kernel(
        _sc_body,
        out_type=jax.ShapeDtypeStruct((NC, N_NODES, D), jnp.float32),
        mesh=mesh,
        scratch_types=(
            [pltpu.VMEM((BLK, D), jnp.float32)] * 3
            + [pltpu.VMEM((BLK // 8, 8 * D), jnp.float32)] * 2
            + [pltpu.VMEM((BLK,), jnp.int32)] * 3
            + [pltpu.VMEM((BLK,), jnp.int32)] * 4
            + [pltpu.VMEM_SHARED((N_NODES, D), jnp.float32)]
            + [pltpu.SemaphoreType.DMA] * 10
        ),
    )
    return f(x, emb, src2, dst2)


def _mlp_body(x_ref, p0_ref, p1_ref, w0_ref, b0_ref, w1_ref, b1_ref, out_ref):
    h = x_ref[...] + p0_ref[...] + p1_ref[...]
    h = jnp.maximum(
        jnp.dot(h, w0_ref[...], preferred_element_type=jnp.float32) + b0_ref[...],
        0.0,
    )
    out_ref[...] = (
        jnp.dot(h, w1_ref[...], preferred_element_type=jnp.float32) + b1_ref[...]
    )


def _mlp(x, p0, p1, w0t, b0r, w1t, b1r):
    rb = 2000
    return pl.pallas_call(
        _mlp_body,
        grid=(N_NODES // rb,),
        in_specs=[
            pl.BlockSpec((rb, D), lambda i: (i, 0)),
            pl.BlockSpec((rb, D), lambda i: (i, 0)),
            pl.BlockSpec((rb, D), lambda i: (i, 0)),
            pl.BlockSpec((D, HID), lambda i: (0, 0)),
            pl.BlockSpec((1, HID), lambda i: (0, 0)),
            pl.BlockSpec((HID, D), lambda i: (0, 0)),
            pl.BlockSpec((1, D), lambda i: (0, 0)),
        ],
        out_specs=pl.BlockSpec((rb, D), lambda i: (i, 0)),
        out_shape=jax.ShapeDtypeStruct((N_NODES, D), jnp.float32),
    )(x, p0, p1, w0t, b0r, w1t, b1r)


def kernel(x, edge_index, edge_attr, lin_w, lin_b, w0, b0, w1, b1):
    src2 = edge_index[0].astype(jnp.int32).reshape(NB, BLK)
    dst2 = edge_index[1].astype(jnp.int32).reshape(NB, BLK)
    attr128 = edge_attr.reshape(E // 8, 8 * 16)
    w_big = jnp.kron(jnp.eye(8, dtype=jnp.float32), lin_w.T)
    b_big = jnp.tile(lin_b, 8).reshape(1, 8 * D)
    emb = _edge_emb(attr128, w_big, b_big)
    partials = _sc_aggregate(x, emb, src2, dst2)
    return _mlp(x, partials[0], partials[1],
                w0.T, b0.reshape(1, HID), w1.T, b1.reshape(1, D))


# speedup vs baseline: 1.6424x; 1.6424x over previous
import functools
import jax
import jax.numpy as jnp
from jax import lax
from jax.experimental import pallas as pl
from jax.experimental.pallas import tpu as pltpu
from jax.experimental.pallas import tpu_sc as plsc

N_NODES = 10000
D = 128
E = 320000
HID = 64

NC, NS = 2, 16
NW = NC * NS
BLK = 64
NB = E // BLK
NBW = NB // NW
NXTRA = NB - NBW * NW
UNROLL = 12
NSUP = NBW // UNROLL
CH = 40
NCHUNK = N_NODES // CH


def _emb_body(attr_ref, w_ref, b_ref, out_ref):
    out_ref[...] = (
        jnp.dot(attr_ref[...], w_ref[...], preferred_element_type=jnp.float32)
        + b_ref[...]
    )


def _edge_emb(attr128, w_big, b_big):
    rb = 800
    er = E // 8
    return pl.pallas_call(
        _emb_body,
        grid=(er // rb,),
        in_specs=[
            pl.BlockSpec((rb, D), lambda i: (i, 0)),
            pl.BlockSpec((D, 8 * D), lambda i: (0, 0)),
            pl.BlockSpec((1, 8 * D), lambda i: (0, 0)),
        ],
        out_specs=pl.BlockSpec((rb, 8 * D), lambda i: (i, 0)),
        out_shape=jax.ShapeDtypeStruct((er, 8 * D), jnp.float32),
    )(attr128, w_big, b_big)


def _sc_body(x_hbm, emb_hbm, src2_hbm, dst2_hbm, out_hbm,
             rows0, rows1, rows2, emb0, emb1,
             is0, is1, is2, id0, id1, id2, id3,
             accum,
             sg0, sg1, sg2, se0, se1, ss0, ss1, si0, si1, si2):
    cid = lax.axis_index("c")
    sid = lax.axis_index("s")
    wid = sid * NC + cid
    rows = (rows0, rows1, rows2)
    embv = (emb0, emb1)
    isrc = (is0, is1, is2)
    idst = (id0, id1, id2, id3)
    sg = (sg0, sg1, sg2)
    se = (se0, se1)
    ss = (ss0, ss1)
    si = (si0, si1, si2)

    zv = jnp.zeros((16,), jnp.float32)

    def zero_body(i, _):
        for j in range(D // 16):
            rows0[i, pl.ds(j * 16, 16)] = zv
        return 0

    lax.fori_loop(0, CH, zero_body, 0)
    nch = jnp.where(sid < NCHUNK - (NCHUNK // NS) * NS, NCHUNK // NS + 1,
                    NCHUNK // NS)

    def zchunk(k, _):
        r0 = pl.multiple_of((sid + k * NS) * CH, 8)
        pltpu.sync_copy(rows0.at[pl.ds(0, CH)], accum.at[pl.ds(r0, CH)])
        return 0

    lax.fori_loop(0, nch, zchunk, 0)
    plsc.subcore_barrier()


    def fire_idx(t, k3, k4):
        blk = t * NW + wid
        pltpu.async_copy(src2_hbm.at[blk], isrc[k3], si[k3])
        pltpu.async_copy(dst2_hbm.at[blk], idst[k4], si[k3])

    def drain_idx(k3):
        pltpu.make_async_copy(src2_hbm.at[0], isrc[k3], si[k3]).wait()
        pltpu.make_async_copy(dst2_hbm.at[0], idst[0], si[k3]).wait()

    def ebase(t):
        return pl.multiple_of((t * NW + wid) * (BLK // 8), 8)

    def fire_gather(t, k3, k2):
        pltpu.async_copy(x_hbm.at[isrc[k3]], rows[k3], sg[k3])
        pltpu.async_copy(emb_hbm.at[pl.ds(ebase(t), BLK // 8)], embv[k2], se[k2])

    def wait_gather(t, k3, k2):
        pltpu.make_async_copy(x_hbm.at[isrc[k3]], rows[k3], sg[k3]).wait()
        pltpu.make_async_copy(emb_hbm.at[pl.ds(ebase(t), BLK // 8)], embv[k2],
                              se[k2]).wait()

    def compute(k3, k2):
        @plsc.parallel_loop(0, BLK, 1, unroll=2)
        def _(i):
            er = lax.shift_right_logical(i, 3)
            base = (i & 7) * D
            es = [embv[k2][er, pl.ds(base + 16 * j, 16)]
                  for j in range(D // 16)]
            rs = [rows[k3][i, pl.ds(16 * j, 16)] for j in range(D // 16)]
            for j in range(D // 16):
                rows[k3][i, pl.ds(16 * j, 16)] = jnp.maximum(rs[j] + es[j], 0.0)

    fire_idx(0, 0, 0)
    fire_idx(1, 1, 1)
    fire_idx(2, 2, 2)
    drain_idx(0)
    fire_gather(0, 0, 0)
    drain_idx(1)
    fire_gather(1, 1, 1)

    def super_body(s, _):
        for u in range(UNROLL):
            t = s * UNROLL + u
            k3, k2, k4 = u % 3, u % 2, u % 4
            wait_gather(t, k3, k2)
            compute(k3, k2)
            pltpu.async_copy(rows[k3], accum.at[idst[k4]], ss[u % 2], add=True)

            def drain_scat():
                pltpu.make_async_copy(rows[(u - 1) % 3],
                                      accum.at[idst[(u - 1) % 4]],
                                      ss[(u - 1) % 2]).wait()

            if u == 0:
                pl.when(s > 0)(drain_scat)
            else:
                drain_scat()

            @pl.when(t + 2 < NBW)
            def _():
                drain_idx((u + 2) % 3)
                fire_gather(t + 2, (u + 2) % 3, k2)

            @pl.when(t + 3 < NBW)
            def _():
                fire_idx(t + 3, u % 3, (u + 3) % 4)
        return 0

    lax.fori_loop(0, NSUP, super_body, 0)
    pltpu.make_async_copy(rows[(NBW - 1) % 3], accum.at[idst[(NBW - 1) % 4]],
                          ss[(NBW - 1) % 2]).wait()

    @pl.when(wid < NXTRA)
    def _():
        blk = NBW * NW + wid
        pltpu.sync_copy(src2_hbm.at[blk], isrc[0])
        pltpu.sync_copy(dst2_hbm.at[blk], idst[0])
        pltpu.async_copy(x_hbm.at[isrc[0]], rows[0], sg[0]).wait()
        pltpu.sync_copy(emb_hbm.at[pl.ds(ebase(NBW), BLK // 8)], embv[0])
        compute(0, 0)
        pltpu.sync_copy(rows[0], accum.at[idst[0]], add=True)

    plsc.subcore_barrier()

    def wchunk(k, _):
        r0 = pl.multiple_of((sid + k * NS) * CH, 8)
        pltpu.sync_copy(accum.at[pl.ds(r0, CH)], out_hbm.at[cid, pl.ds(r0, CH)])
        return 0

    lax.fori_loop(0, nch, wchunk, 0)


def _sc_aggregate(x, emb, src2, dst2):
    mesh = plsc.VectorSubcoreMesh(core_axis_name="c", subcore_axis_name="s")
    f = pl.kernel(
        _sc_body,
        out_type=jax.ShapeDtypeStruct((NC, N_NODES, D), jnp.float32),
        mesh=mesh,
        scratch_types=(
            [pltpu.VMEM((BLK, D), jnp.float32)] * 3
            + [pltpu.VMEM((BLK // 8, 8 * D), jnp.float32)] * 2
            + [pltpu.VMEM((BLK,), jnp.int32)] * 3
            + [pltpu.VMEM((BLK,), jnp.int32)] * 4
            + [pltpu.VMEM_SHARED((N_NODES, D), jnp.float32)]
            + [pltpu.SemaphoreType.DMA] * 10
        ),
    )
    return f(x, emb, src2, dst2)


def _mlp_body(x_ref, p0_ref, p1_ref, w0_ref, b0_ref, w1_ref, b1_ref, out_ref):
    h = x_ref[...] + p0_ref[...] + p1_ref[...]
    h = jnp.maximum(
        jnp.dot(h, w0_ref[...], preferred_element_type=jnp.float32) + b0_ref[...],
        0.0,
    )
    out_ref[...] = (
        jnp.dot(h, w1_ref[...], preferred_element_type=jnp.float32) + b1_ref[...]
    )


def _mlp(x, p0, p1, w0t, b0r, w1t, b1r):
    rb = 2000
    return pl.pallas_call(
        _mlp_body,
        grid=(N_NODES // rb,),
        in_specs=[
            pl.BlockSpec((rb, D), lambda i: (i, 0)),
            pl.BlockSpec((rb, D), lambda i: (i, 0)),
            pl.BlockSpec((rb, D), lambda i: (i, 0)),
            pl.BlockSpec((D, HID), lambda i: (0, 0)),
            pl.BlockSpec((1, HID), lambda i: (0, 0)),
            pl.BlockSpec((HID, D), lambda i: (0, 0)),
            pl.BlockSpec((1, D), lambda i: (0, 0)),
        ],
        out_specs=pl.BlockSpec((rb, D), lambda i: (i, 0)),
        out_shape=jax.ShapeDtypeStruct((N_NODES, D), jnp.float32),
    )(x, p0, p1, w0t, b0r, w1t, b1r)


def kernel(x, edge_index, edge_attr, lin_w, lin_b, w0, b0, w1, b1):
    src2 = edge_index[0].astype(jnp.int32).reshape(NB, BLK)
    dst2 = edge_index[1].astype(jnp.int32).reshape(NB, BLK)
    attr128 = edge_attr.reshape(E // 8, 8 * 16)
    w_big = jnp.kron(jnp.eye(8, dtype=jnp.float32), lin_w.T)
    b_big = jnp.tile(lin_b, 8).reshape(1, 8 * D)
    emb = _edge_emb(attr128, w_big, b_big)
    partials = _sc_aggregate(x, emb, src2, dst2)
    return _mlp(x, partials[0], partials[1],
                w0.T, b0.reshape(1, HID), w1.T, b1.reshape(1, D))


# speedup vs baseline: 1.6573x; 1.0091x over previous
import functools
import jax
import jax.numpy as jnp
from jax import lax
from jax.experimental import pallas as pl
from jax.experimental.pallas import tpu as pltpu
from jax.experimental.pallas import tpu_sc as plsc

N_NODES = 10000
D = 128
E = 320000
HID = 64

NC, NS = 2, 16
NW = NC * NS
BLK = 64
NB = E // BLK
NBW = NB // NW
NXTRA = NB - NBW * NW
UNROLL = 12
NSUP = NBW // UNROLL
CH = 40
NCHUNK = N_NODES // CH


def _emb_body(attr_ref, w_ref, b_ref, out_ref):
    out_ref[...] = (
        jnp.dot(attr_ref[...], w_ref[...], preferred_element_type=jnp.float32)
        + b_ref[...]
    )


def _edge_emb(attr128, w_big, b_big, rb):
    er = attr128.shape[0]
    return pl.pallas_call(
        _emb_body,
        grid=(er // rb,),
        in_specs=[
            pl.BlockSpec((rb, D), lambda i: (i, 0)),
            pl.BlockSpec((D, 8 * D), lambda i: (0, 0)),
            pl.BlockSpec((1, 8 * D), lambda i: (0, 0)),
        ],
        out_specs=pl.BlockSpec((rb, 8 * D), lambda i: (i, 0)),
        out_shape=jax.ShapeDtypeStruct((er, 8 * D), jnp.float32),
    )(attr128, w_big, b_big)


def _make_sc_body(t0, t1, extra, eoff):
    def body(x_hbm, emb_hbm, src2_hbm, dst2_hbm, out_hbm,
             rows0, rows1, rows2, emb0, emb1,
             is0, is1, is2, id0, id1, id2, id3,
             accum,
             sg0, sg1, sg2, se0, se1, ss0, ss1, si0, si1, si2):
        _sc_body_impl(t0, t1, extra, eoff,
                      x_hbm, emb_hbm, src2_hbm, dst2_hbm, out_hbm,
                      rows0, rows1, rows2, emb0, emb1,
                      is0, is1, is2, id0, id1, id2, id3, accum,
                      sg0, sg1, sg2, se0, se1, ss0, ss1, si0, si1, si2)
    return body


def _sc_body_impl(t0, t1, extra, eoff,
                  x_hbm, emb_hbm, src2_hbm, dst2_hbm, out_hbm,
                  rows0, rows1, rows2, emb0, emb1,
                  is0, is1, is2, id0, id1, id2, id3,
                  accum,
                  sg0, sg1, sg2, se0, se1, ss0, ss1, si0, si1, si2):
    cid = lax.axis_index("c")
    sid = lax.axis_index("s")
    wid = sid * NC + cid
    rows = (rows0, rows1, rows2)
    embv = (emb0, emb1)
    isrc = (is0, is1, is2)
    idst = (id0, id1, id2, id3)
    sg = (sg0, sg1, sg2)
    se = (se0, se1)
    ss = (ss0, ss1)
    si = (si0, si1, si2)

    zv = jnp.zeros((16,), jnp.float32)

    def zero_body(i, _):
        for j in range(D // 16):
            rows0[i, pl.ds(j * 16, 16)] = zv
        return 0

    lax.fori_loop(0, CH, zero_body, 0)
    nch = jnp.where(sid < NCHUNK - (NCHUNK // NS) * NS, NCHUNK // NS + 1,
                    NCHUNK // NS)

    def zchunk(k, _):
        r0 = pl.multiple_of((sid + k * NS) * CH, 8)
        pltpu.sync_copy(rows0.at[pl.ds(0, CH)], accum.at[pl.ds(r0, CH)])
        return 0

    lax.fori_loop(0, nch, zchunk, 0)
    plsc.subcore_barrier()


    def fire_idx(t, k3, k4):
        blk = t * NW + wid
        pltpu.async_copy(src2_hbm.at[blk], isrc[k3], si[k3])
        pltpu.async_copy(dst2_hbm.at[blk], idst[k4], si[k3])

    def drain_idx(k3):
        pltpu.make_async_copy(src2_hbm.at[0], isrc[k3], si[k3]).wait()
        pltpu.make_async_copy(dst2_hbm.at[0], idst[0], si[k3]).wait()

    def ebase(t):
        return pl.multiple_of((t * NW + wid) * (BLK // 8) - eoff, 8)

    def fire_gather(t, k3, k2):
        pltpu.async_copy(x_hbm.at[isrc[k3]], rows[k3], sg[k3])
        pltpu.async_copy(emb_hbm.at[pl.ds(ebase(t), BLK // 8)], embv[k2], se[k2])

    def wait_gather(t, k3, k2):
        pltpu.make_async_copy(x_hbm.at[isrc[k3]], rows[k3], sg[k3]).wait()
        pltpu.make_async_copy(emb_hbm.at[pl.ds(ebase(t), BLK // 8)], embv[k2],
                              se[k2]).wait()

    def compute(k3, k2):
        @plsc.parallel_loop(0, BLK, 1, unroll=2)
        def _(i):
            er = lax.shift_right_logical(i, 3)
            base = (i & 7) * D
            es = [embv[k2][er, pl.ds(base + 16 * j, 16)]
                  for j in range(D // 16)]
            rs = [rows[k3][i, pl.ds(16 * j, 16)] for j in range(D // 16)]
            for j in range(D // 16):
                rows[k3][i, pl.ds(16 * j, 16)] = jnp.maximum(rs[j] + es[j], 0.0)

    fire_idx(t0, 0, 0)
    fire_idx(t0 + 1, 1, 1)
    fire_idx(t0 + 2, 2, 2)
    drain_idx(0)
    fire_gather(t0, 0, 0)
    drain_idx(1)
    fire_gather(t0 + 1, 1, 1)

    def super_body(s, _):
        for u in range(UNROLL):
            t = s * UNROLL + u
            k3, k2, k4 = u % 3, u % 2, u % 4
            wait_gather(t, k3, k2)
            compute(k3, k2)
            pltpu.async_copy(rows[k3], accum.at[idst[k4]], ss[u % 2], add=True)

            def drain_scat():
                pltpu.make_async_copy(rows[(u - 1) % 3],
                                      accum.at[idst[(u - 1) % 4]],
                                      ss[(u - 1) % 2]).wait()

            if u == 0:
                pl.when(s > t0 // UNROLL)(drain_scat)
            else:
                drain_scat()

            @pl.when(t + 2 < t1)
            def _():
                drain_idx((u + 2) % 3)
                fire_gather(t + 2, (u + 2) % 3, k2)

            @pl.when(t + 3 < t1)
            def _():
                fire_idx(t + 3, u % 3, (u + 3) % 4)
        return 0

    lax.fori_loop(t0 // UNROLL, t1 // UNROLL, super_body, 0)
    pltpu.make_async_copy(rows[(t1 - 1) % 3], accum.at[idst[(t1 - 1) % 4]],
                          ss[(t1 - 1) % 2]).wait()

    if extra:
        @pl.when(wid < NXTRA)
        def _():
            blk = NBW * NW + wid
            pltpu.sync_copy(src2_hbm.at[blk], isrc[0])
            pltpu.sync_copy(dst2_hbm.at[blk], idst[0])
            pltpu.async_copy(x_hbm.at[isrc[0]], rows[0], sg[0]).wait()
            pltpu.sync_copy(emb_hbm.at[pl.ds(ebase(NBW), BLK // 8)], embv[0])
            compute(0, 0)
            pltpu.sync_copy(rows[0], accum.at[idst[0]], add=True)

    plsc.subcore_barrier()

    def wchunk(k, _):
        r0 = pl.multiple_of((sid + k * NS) * CH, 8)
        pltpu.sync_copy(accum.at[pl.ds(r0, CH)], out_hbm.at[cid, pl.ds(r0, CH)])
        return 0

    lax.fori_loop(0, nch, wchunk, 0)


def _sc_aggregate(x, emb, src2, dst2, t0, t1, extra):
    mesh = plsc.VectorSubcoreMesh(core_axis_name="c", subcore_axis_name="s")
    f = pl.kernel(
        _make_sc_body(t0, t1, extra, t0 * NW * (BLK // 8)),
        out_type=jax.ShapeDtypeStruct((NC, N_NODES, D), jnp.float32),
        mesh=mesh,
        scratch_types=(
            [pltpu.VMEM((BLK, D), jnp.float32)] * 3
            + [pltpu.VMEM((BLK // 8, 8 * D), jnp.float32)] * 2
            + [pltpu.VMEM((BLK,), jnp.int32)] * 3
            + [pltpu.VMEM((BLK,), jnp.int32)] * 4
            + [pltpu.VMEM_SHARED((N_NODES, D), jnp.float32)]
            + [pltpu.SemaphoreType.DMA] * 10
        ),
    )
    return f(x, emb, src2, dst2)


def _mlp_body(x_ref, p0_ref, p1_ref, p2_ref, p3_ref,
              w0_ref, b0_ref, w1_ref, b1_ref, out_ref):
    h = (x_ref[...] + p0_ref[...] + p1_ref[...]
         + p2_ref[...] + p3_ref[...])
    h = jnp.maximum(
        jnp.dot(h, w0_ref[...], preferred_element_type=jnp.float32) + b0_ref[...],
        0.0,
    )
    out_ref[...] = (
        jnp.dot(h, w1_ref[...], preferred_element_type=jnp.float32) + b1_ref[...]
    )


def _mlp(x, p0, p1, p2, p3, w0t, b0r, w1t, b1r):
    rb = 2000
    return pl.pallas_call(
        _mlp_body,
        grid=(N_NODES // rb,),
        in_specs=[
            pl.BlockSpec((rb, D), lambda i: (i, 0)),
            pl.BlockSpec((rb, D), lambda i: (i, 0)),
            pl.BlockSpec((rb, D), lambda i: (i, 0)),
            pl.BlockSpec((rb, D), lambda i: (i, 0)),
            pl.BlockSpec((rb, D), lambda i: (i, 0)),
            pl.BlockSpec((D, HID), lambda i: (0, 0)),
            pl.BlockSpec((1, HID), lambda i: (0, 0)),
            pl.BlockSpec((HID, D), lambda i: (0, 0)),
            pl.BlockSpec((1, D), lambda i: (0, 0)),
        ],
        out_specs=pl.BlockSpec((rb, D), lambda i: (i, 0)),
        out_shape=jax.ShapeDtypeStruct((N_NODES, D), jnp.float32),
    )(x, p0, p1, p2, p3, w0t, b0r, w1t, b1r)


TSPLIT = 60
EDGES_A = TSPLIT * NW * BLK


def kernel(x, edge_index, edge_attr, lin_w, lin_b, w0, b0, w1, b1):
    src2 = edge_index[0].astype(jnp.int32).reshape(NB, BLK)
    dst2 = edge_index[1].astype(jnp.int32).reshape(NB, BLK)
    w_big = jnp.kron(jnp.eye(8, dtype=jnp.float32), lin_w.T)
    b_big = jnp.tile(lin_b, 8).reshape(1, 8 * D)
    attr_a = edge_attr[:EDGES_A].reshape(EDGES_A // 8, 8 * 16)
    attr_b = edge_attr[EDGES_A:].reshape((E - EDGES_A) // 8, 8 * 16)
    emb_a = _edge_emb(attr_a, w_big, b_big, 640)
    emb_b = _edge_emb(attr_b, w_big, b_big, 448)
    pa = _sc_aggregate(x, emb_a, src2, dst2, 0, TSPLIT, False)
    pb = _sc_aggregate(x, emb_b, src2, dst2, TSPLIT, NBW, True)
    return _mlp(x, pa[0], pa[1], pb[0], pb[1],
                w0.T, b0.reshape(1, HID), w1.T, b1.reshape(1, D))
